# baseline (speedup 1.0000x reference)
import math

import jax
import jax.numpy as jnp
from jax.experimental import pallas as pl
from jax.experimental.pallas import tpu as pltpu

_BASE_GSD = 10.0
_D = 192
_LN1E4_OVER = math.log(10000.0) / (_D // 4)
_HBLK = 2


def _tc_body(months_ref, gsd_ref, pos_ref, mtab_ref, ch7_ref, ch_sp_ref,
             ch_t_ref, ch_st_ref, s_t_hbm, sp_ref, t_ref, st_ref,
             s_t_out_hbm, sp_out_ref, t_out_ref, st_out_ref,
             inbuf, outbuf, emb_ref, spat_pad_ref, in_sem, out_sem):
    i = pl.program_id(0)
    nh = 8 // _HBLK
    n = pl.num_programs(0)
    b = i // nh
    hh = i % nh
    h, w, t = 8, 8, 12
    d = _D
    slot = jax.lax.rem(i, 2)

    def in_dma(j, s):
        return pltpu.make_async_copy(
            s_t_hbm.at[j // nh, pl.ds(_HBLK * (j % nh), _HBLK)],
            inbuf.at[s], in_sem.at[s])

    def out_dma(j, s):
        return pltpu.make_async_copy(
            outbuf.at[s],
            s_t_out_hbm.at[j // nh, pl.ds(_HBLK * (j % nh), _HBLK)],
            out_sem.at[s])

    @pl.when(i == 0)
    def _prologue():
        in_dma(0, 0).start()
        res = gsd_ref[0]
        wc = jax.lax.broadcasted_iota(jnp.int32, (h, w, d), 1).astype(jnp.float32)
        hc = jax.lax.broadcasted_iota(jnp.int32, (h, w, d), 0).astype(jnp.float32)
        col = jax.lax.broadcasted_iota(jnp.int32, (h, w, d), 2)
        p = jnp.where(col < d // 2, wc, hc) * res
        k = col % (d // 2)
        kk = (k % (d // 4)).astype(jnp.float32)
        omega = jnp.exp(kk * (-_LN1E4_OVER))
        phase = jnp.where(k < d // 4, 0.0, 0.5 * jnp.pi).astype(jnp.float32)
        spatial = jnp.sin(p * omega + phase)
        spat_pad_ref[...] = jnp.concatenate(
            [jnp.zeros((h, w, 3 * d), jnp.float32), spatial], axis=-1)

    @pl.when(i + 1 < n)
    def _prefetch():
        in_dma(i + 1, 1 - slot).start()

    @pl.when(hh == 0)
    def _per_batch():
        pos12 = pos_ref[...]
        mon12 = jnp.concatenate(
            [mtab_ref[pl.ds(months_ref[b, j], 1), :] for j in range(t)],
            axis=0)
        emb_ref[...] = jnp.concatenate(
            [jnp.broadcast_to(ch7_ref[...][None], (t, 7, d)),
             jnp.broadcast_to(pos12[:, None, :], (t, 7, d)),
             jnp.broadcast_to(mon12[:, None, :], (t, 7, d)),
             jnp.zeros((t, 7, d), jnp.float32)], axis=-1)

        emb_t = jnp.concatenate(
            [jnp.broadcast_to(ch_t_ref[...][None], (t, 3, d)),
             jnp.broadcast_to(pos12[:, None, :], (t, 3, d)),
             jnp.broadcast_to(mon12[:, None, :], (t, 3, d)),
             jnp.zeros((t, 3, d), jnp.float32)], axis=-1)
        t_out_ref[0] = t_ref[0] + emb_t

        st_row = jnp.concatenate(
            [ch_st_ref[...], jnp.zeros((3, 3 * d), jnp.float32)], axis=-1)
        st_out_ref[0] = st_ref[0] + st_row

    spat = spat_pad_ref[pl.ds(_HBLK * hh, _HBLK)]

    sp_row = jnp.concatenate(
        [ch_sp_ref[...], jnp.zeros((3, 3 * d), jnp.float32)], axis=-1)
    sp_out_ref[0] = sp_ref[0] + sp_row[None, None] + spat[:, :, None, :]

    in_dma(i, slot).wait()

    @pl.when(i >= 2)
    def _drain_prev():
        out_dma(i - 2, slot).wait()

    outbuf[slot] = (inbuf[slot] + emb_ref[...][None, None]
                    + spat[:, :, None, None, :])
    out_dma(i, slot).start()

    @pl.when(i == n - 1)
    def _epilogue():
        out_dma(i - 1, 1 - slot).wait()
        out_dma(i, slot).wait()


def kernel(s_t_x, sp_x, t_x, st_x, months, patch_size, input_res, pos_embed_p,
           month_tab, s_t_channel_embed, sp_channel_embed, t_channel_embed,
           st_channel_embed):
    b, h, w, t, g7, e = s_t_x.shape
    gsd = (jnp.asarray(input_res, jnp.float32)
           * jnp.asarray(patch_size, jnp.float32) / _BASE_GSD).reshape(1)

    nh = h // _HBLK
    grid = (b * nh,)
    full = lambda a: pl.BlockSpec(a.shape, lambda i: (0,) * a.ndim)
    hbm = pl.BlockSpec(memory_space=pltpu.MemorySpace.HBM)
    in_specs = [
        pl.BlockSpec(months.shape, lambda i: (0, 0),
                     memory_space=pltpu.SMEM),
        pl.BlockSpec((1,), lambda i: (0,), memory_space=pltpu.SMEM),
        full(pos_embed_p[:t]), full(month_tab), full(s_t_channel_embed),
        full(sp_channel_embed), full(t_channel_embed), full(st_channel_embed),
        hbm,
        pl.BlockSpec((1, _HBLK, w, 3, e),
                     lambda i: (i // nh, i % nh, 0, 0, 0)),
        pl.BlockSpec((1, t, 3, e), lambda i: (i // nh, 0, 0, 0)),
        pl.BlockSpec((1, 3, e), lambda i: (i // nh, 0, 0)),
    ]
    out_specs = [
        hbm,
        pl.BlockSpec((1, _HBLK, w, 3, e),
                     lambda i: (i // nh, i % nh, 0, 0, 0)),
        pl.BlockSpec((1, t, 3, e), lambda i: (i // nh, 0, 0, 0)),
        pl.BlockSpec((1, 3, e), lambda i: (i // nh, 0, 0)),
    ]
    out_shapes = [
        jax.ShapeDtypeStruct(s_t_x.shape, jnp.float32),
        jax.ShapeDtypeStruct(sp_x.shape, jnp.float32),
        jax.ShapeDtypeStruct(t_x.shape, jnp.float32),
        jax.ShapeDtypeStruct(st_x.shape, jnp.float32),
    ]
    outs = pl.pallas_call(
        _tc_body,
        grid=grid,
        in_specs=in_specs,
        out_specs=out_specs,
        out_shape=out_shapes,
        scratch_shapes=[
            pltpu.VMEM((2, _HBLK, w, t, g7, e), jnp.float32),
            pltpu.VMEM((2, _HBLK, w, t, g7, e), jnp.float32),
            pltpu.VMEM((t, g7, e), jnp.float32),
            pltpu.VMEM((h, w, e), jnp.float32),
            pltpu.SemaphoreType.DMA((2,)),
            pltpu.SemaphoreType.DMA((2,)),
        ],
        compiler_params=pltpu.CompilerParams(
            dimension_semantics=("arbitrary",)),
    )(months, gsd, pos_embed_p[:t], month_tab, s_t_channel_embed,
      sp_channel_embed, t_channel_embed, st_channel_embed,
      s_t_x, sp_x, t_x, st_x)
    return tuple(outs)


# speedup vs baseline: 3.4669x; 3.4669x over previous
import math

import jax
import jax.numpy as jnp
from jax.experimental import pallas as pl
from jax.experimental.pallas import tpu as pltpu

_BASE_GSD = 10.0
_D = 192
_LN1E4_OVER = math.log(10000.0) / (_D // 4)
_HBLK = 2


def _tc_body(months_ref, gsd_ref, pos_ref, mtab_ref, ch7_ref, ch_sp_ref,
             ch_t_ref, ch_st_ref, xt_ref, spt_ref, tt_ref, stt_ref,
             xt_out_ref, spt_out_ref, tt_out_ref, stt_out_ref,
             emb_ref, spat_pad_ref):
    b = pl.program_id(0)
    hh = pl.program_id(1)
    h, w, t = 8, 8, 12
    d = _D

    @pl.when((b == 0) & (hh == 0))
    def _first_step():
        res = gsd_ref[0]
        wc = jax.lax.broadcasted_iota(jnp.int32, (h, w, d), 1).astype(jnp.float32)
        hc = jax.lax.broadcasted_iota(jnp.int32, (h, w, d), 0).astype(jnp.float32)
        col = jax.lax.broadcasted_iota(jnp.int32, (h, w, d), 2)
        p = jnp.where(col < d // 2, wc, hc) * res
        k = col % (d // 2)
        kk = (k % (d // 4)).astype(jnp.float32)
        omega = jnp.exp(kk * (-_LN1E4_OVER))
        phase = jnp.where(k < d // 4, 0.0, 0.5 * jnp.pi).astype(jnp.float32)
        spatial = jnp.sin(p * omega + phase)
        spat_pad_ref[...] = jnp.concatenate(
            [jnp.zeros((h, w, 3 * d), jnp.float32), spatial], axis=-1)

        pos12 = pos_ref[...]
        mon_tb = jnp.concatenate(
            [mtab_ref[pl.ds(months_ref[bb, tj], 1), :]
             for tj in range(t) for bb in range(4)], axis=0
        ).reshape(t, 4, d)
        emb_t = jnp.concatenate(
            [jnp.broadcast_to(ch_t_ref[...][None, :, None, :], (t, 3, 4, d)),
             jnp.broadcast_to(pos12[:, None, None, :], (t, 3, 4, d)),
             jnp.broadcast_to(mon_tb[:, None, :, :], (t, 3, 4, d)),
             jnp.zeros((t, 3, 4, d), jnp.float32)], axis=-1)
        tt_out_ref[...] = tt_ref[...] + emb_t

        st_row = jnp.concatenate(
            [ch_st_ref[...], jnp.zeros((3, 3 * d), jnp.float32)], axis=-1)
        stt_out_ref[...] = stt_ref[...] + st_row[:, None, :]

    @pl.when(hh == 0)
    def _per_batch():
        pos12 = pos_ref[...]
        mon12 = jnp.concatenate(
            [mtab_ref[pl.ds(months_ref[b, j], 1), :] for j in range(t)],
            axis=0)
        emb_ref[...] = jnp.concatenate(
            [jnp.broadcast_to(ch7_ref[...][None], (t, 7, d)),
             jnp.broadcast_to(pos12[:, None, :], (t, 7, d)),
             jnp.broadcast_to(mon12[:, None, :], (t, 7, d)),
             jnp.zeros((t, 7, d), jnp.float32)], axis=-1)

    spat = spat_pad_ref[pl.ds(_HBLK * hh, _HBLK)]

    xt_out_ref[0] = (xt_ref[0] + emb_ref[...][None, :, :, None, :]
                     + spat[:, None, None, :, :])

    sp_row = jnp.concatenate(
        [ch_sp_ref[...], jnp.zeros((3, 3 * d), jnp.float32)], axis=-1)
    spt_out_ref[0] = (spt_ref[0] + sp_row[None, :, None, :]
                      + spat[:, None, :, :])


def kernel(s_t_x, sp_x, t_x, st_x, months, patch_size, input_res, pos_embed_p,
           month_tab, s_t_channel_embed, sp_channel_embed, t_channel_embed,
           st_channel_embed):
    b, h, w, t, g7, e = s_t_x.shape
    gsd = (jnp.asarray(input_res, jnp.float32)
           * jnp.asarray(patch_size, jnp.float32) / _BASE_GSD).reshape(1)

    xt = jnp.transpose(s_t_x, (0, 1, 3, 4, 2, 5))
    spt = jnp.transpose(sp_x, (0, 1, 3, 2, 4))
    tt = jnp.transpose(t_x, (1, 2, 0, 3))
    stt = jnp.transpose(st_x, (1, 0, 2))

    nh = h // _HBLK
    grid = (b, nh)
    full = lambda a: pl.BlockSpec(a.shape, lambda bi, hi: (0,) * a.ndim)
    in_specs = [
        pl.BlockSpec(months.shape, lambda bi, hi: (0, 0),
                     memory_space=pltpu.SMEM),
        pl.BlockSpec((1,), lambda bi, hi: (0,), memory_space=pltpu.SMEM),
        full(pos_embed_p[:t]), full(month_tab), full(s_t_channel_embed),
        full(sp_channel_embed), full(t_channel_embed), full(st_channel_embed),
        pl.BlockSpec((1, _HBLK, t, g7, w, e),
                     lambda bi, hi: (bi, hi, 0, 0, 0, 0)),
        pl.BlockSpec((1, _HBLK, 3, w, e), lambda bi, hi: (bi, hi, 0, 0, 0)),
        full(tt), full(stt),
    ]
    out_specs = [
        pl.BlockSpec((1, _HBLK, t, g7, w, e),
                     lambda bi, hi: (bi, hi, 0, 0, 0, 0)),
        pl.BlockSpec((1, _HBLK, 3, w, e), lambda bi, hi: (bi, hi, 0, 0, 0)),
        full(tt), full(stt),
    ]
    out_shapes = [
        jax.ShapeDtypeStruct(xt.shape, jnp.float32),
        jax.ShapeDtypeStruct(spt.shape, jnp.float32),
        jax.ShapeDtypeStruct(tt.shape, jnp.float32),
        jax.ShapeDtypeStruct(stt.shape, jnp.float32),
    ]
    outs = pl.pallas_call(
        _tc_body,
        grid=grid,
        in_specs=in_specs,
        out_specs=out_specs,
        out_shape=out_shapes,
        scratch_shapes=[pltpu.VMEM((t, g7, e), jnp.float32),
                        pltpu.VMEM((h, w, e), jnp.float32)],
        compiler_params=pltpu.CompilerParams(
            dimension_semantics=("arbitrary", "arbitrary")),
    )(months, gsd, pos_embed_p[:t], month_tab, s_t_channel_embed,
      sp_channel_embed, t_channel_embed, st_channel_embed,
      xt, spt, tt, stt)
    s_t_o, sp_o, t_o, st_o = outs
    return (jnp.transpose(s_t_o, (0, 1, 4, 2, 3, 5)),
            jnp.transpose(sp_o, (0, 1, 3, 2, 4)),
            jnp.transpose(t_o, (2, 0, 1, 3)),
            jnp.transpose(st_o, (1, 0, 2)))


# speedup vs baseline: 3.5630x; 1.0277x over previous
import math

import jax
import jax.numpy as jnp
from jax.experimental import pallas as pl
from jax.experimental.pallas import tpu as pltpu

_BASE_GSD = 10.0
_D = 192
_LN1E4_OVER = math.log(10000.0) / (_D // 4)
_HBLK = 4


def _tc_body(months_ref, gsd_ref, pos_ref, mtab_ref, ch7_ref, ch_sp_ref,
             ch_t_ref, ch_st_ref, xt_ref, spt_ref, tt_ref, stt_ref,
             xt_out_ref, spt_out_ref, tt_out_ref, stt_out_ref,
             emb_ref, spat_pad_ref):
    b = pl.program_id(0)
    hh = pl.program_id(1)
    h, w, t = 8, 8, 12
    d = _D

    @pl.when((b == 0) & (hh == 0))
    def _first_step():
        res = gsd_ref[0]
        wc = jax.lax.broadcasted_iota(jnp.int32, (h, w, d), 1).astype(jnp.float32)
        hc = jax.lax.broadcasted_iota(jnp.int32, (h, w, d), 0).astype(jnp.float32)
        col = jax.lax.broadcasted_iota(jnp.int32, (h, w, d), 2)
        p = jnp.where(col < d // 2, wc, hc) * res
        k = col % (d // 2)
        kk = (k % (d // 4)).astype(jnp.float32)
        omega = jnp.exp(kk * (-_LN1E4_OVER))
        phase = jnp.where(k < d // 4, 0.0, 0.5 * jnp.pi).astype(jnp.float32)
        spatial = jnp.sin(p * omega + phase)
        spat_pad_ref[...] = jnp.concatenate(
            [jnp.zeros((h, w, 3 * d), jnp.float32), spatial], axis=-1)

        pos12 = pos_ref[...]
        mon_tb = jnp.concatenate(
            [mtab_ref[pl.ds(months_ref[bb, tj], 1), :]
             for tj in range(t) for bb in range(4)], axis=0
        ).reshape(t, 4, d)
        emb_t = jnp.concatenate(
            [jnp.broadcast_to(ch_t_ref[...][None, :, None, :], (t, 3, 4, d)),
             jnp.broadcast_to(pos12[:, None, None, :], (t, 3, 4, d)),
             jnp.broadcast_to(mon_tb[:, None, :, :], (t, 3, 4, d)),
             jnp.zeros((t, 3, 4, d), jnp.float32)], axis=-1)
        tt_out_ref[...] = tt_ref[...] + emb_t

        st_row = jnp.concatenate(
            [ch_st_ref[...], jnp.zeros((3, 3 * d), jnp.float32)], axis=-1)
        stt_out_ref[...] = stt_ref[...] + st_row[:, None, :]

    @pl.when(hh == 0)
    def _per_batch():
        pos12 = pos_ref[...]
        mon12 = jnp.concatenate(
            [mtab_ref[pl.ds(months_ref[b, j], 1), :] for j in range(t)],
            axis=0)
        emb_ref[...] = jnp.concatenate(
            [jnp.broadcast_to(ch7_ref[...][None], (t, 7, d)),
             jnp.broadcast_to(pos12[:, None, :], (t, 7, d)),
             jnp.broadcast_to(mon12[:, None, :], (t, 7, d)),
             jnp.zeros((t, 7, d), jnp.float32)], axis=-1)

    spat = spat_pad_ref[pl.ds(_HBLK * hh, _HBLK)]

    xt_out_ref[0] = (xt_ref[0] + emb_ref[...][None, :, :, None, :]
                     + spat[:, None, None, :, :])

    sp_row = jnp.concatenate(
        [ch_sp_ref[...], jnp.zeros((3, 3 * d), jnp.float32)], axis=-1)
    spt_out_ref[0] = (spt_ref[0] + sp_row[None, :, None, :]
                      + spat[:, None, :, :])


def kernel(s_t_x, sp_x, t_x, st_x, months, patch_size, input_res, pos_embed_p,
           month_tab, s_t_channel_embed, sp_channel_embed, t_channel_embed,
           st_channel_embed):
    b, h, w, t, g7, e = s_t_x.shape
    gsd = (jnp.asarray(input_res, jnp.float32)
           * jnp.asarray(patch_size, jnp.float32) / _BASE_GSD).reshape(1)

    xt = jnp.transpose(s_t_x, (0, 1, 3, 4, 2, 5))
    spt = jnp.transpose(sp_x, (0, 1, 3, 2, 4))
    tt = jnp.transpose(t_x, (1, 2, 0, 3))
    stt = jnp.transpose(st_x, (1, 0, 2))

    nh = h // _HBLK
    grid = (b, nh)
    full = lambda a: pl.BlockSpec(a.shape, lambda bi, hi: (0,) * a.ndim)
    in_specs = [
        pl.BlockSpec(months.shape, lambda bi, hi: (0, 0),
                     memory_space=pltpu.SMEM),
        pl.BlockSpec((1,), lambda bi, hi: (0,), memory_space=pltpu.SMEM),
        full(pos_embed_p[:t]), full(month_tab), full(s_t_channel_embed),
        full(sp_channel_embed), full(t_channel_embed), full(st_channel_embed),
        pl.BlockSpec((1, _HBLK, t, g7, w, e),
                     lambda bi, hi: (bi, hi, 0, 0, 0, 0)),
        pl.BlockSpec((1, _HBLK, 3, w, e), lambda bi, hi: (bi, hi, 0, 0, 0)),
        full(tt), full(stt),
    ]
    out_specs = [
        pl.BlockSpec((1, _HBLK, t, g7, w, e),
                     lambda bi, hi: (bi, hi, 0, 0, 0, 0)),
        pl.BlockSpec((1, _HBLK, 3, w, e), lambda bi, hi: (bi, hi, 0, 0, 0)),
        full(tt), full(stt),
    ]
    out_shapes = [
        jax.ShapeDtypeStruct(xt.shape, jnp.float32),
        jax.ShapeDtypeStruct(spt.shape, jnp.float32),
        jax.ShapeDtypeStruct(tt.shape, jnp.float32),
        jax.ShapeDtypeStruct(stt.shape, jnp.float32),
    ]
    outs = pl.pallas_call(
        _tc_body,
        grid=grid,
        in_specs=in_specs,
        out_specs=out_specs,
        out_shape=out_shapes,
        scratch_shapes=[pltpu.VMEM((t, g7, e), jnp.float32),
                        pltpu.VMEM((h, w, e), jnp.float32)],
        compiler_params=pltpu.CompilerParams(
            dimension_semantics=("arbitrary", "arbitrary")),
    )(months, gsd, pos_embed_p[:t], month_tab, s_t_channel_embed,
      sp_channel_embed, t_channel_embed, st_channel_embed,
      xt, spt, tt, stt)
    s_t_o, sp_o, t_o, st_o = outs
    return (jnp.transpose(s_t_o, (0, 1, 4, 2, 3, 5)),
            jnp.transpose(sp_o, (0, 1, 3, 2, 4)),
            jnp.transpose(t_o, (2, 0, 1, 3)),
            jnp.transpose(st_o, (1, 0, 2)))
